# baseline (speedup 1.0000x reference)
import functools

import jax
import jax.numpy as jnp
from jax import lax
from jax.experimental import pallas as pl
from jax.experimental.pallas import tpu as pltpu
from jax.experimental.pallas import tpu_sc as plsc

EPS = 1e-5

NC = 2
NS = 16
CHUNK = 128
NBUF = 3
ZCH = 80


def _matmul_body(x_ref, w_ref, h_ref):
    h_ref[...] = jnp.dot(x_ref[...], w_ref[...],
                         preferred_element_type=jnp.float32)


def _matmul(x, W, block_rows):
    n, d = x.shape
    return pl.pallas_call(
        _matmul_body,
        grid=(n // block_rows,),
        in_specs=[
            pl.BlockSpec((block_rows, d), lambda i: (i, 0)),
            pl.BlockSpec((d, d), lambda i: (0, 0)),
        ],
        out_specs=pl.BlockSpec((block_rows, d), lambda i: (i, 0)),
        out_shape=jax.ShapeDtypeStruct((n, d), jnp.float32),
    )(x, W)


def _make_sc_agg(n, d, e):
    per_w = e // (NC * NS)
    chunks = per_w // CHUNK
    tailn = per_w - chunks * CHUNK
    assert tailn % 8 == 0 and chunks % NBUF == 0
    nzch = n // ZCH
    max_per_tile = (nzch + NS - 1) // NS

    mesh = plsc.VectorSubcoreMesh(core_axis_name="c", subcore_axis_name="s")

    @functools.partial(
        pl.kernel,
        mesh=mesh,
        out_type=jax.ShapeDtypeStruct((NC, n, d), jnp.float32),
        compiler_params=pltpu.CompilerParams(use_tc_tiling_on_sc=False),
        scratch_types=[
            [pltpu.VMEM((CHUNK,), jnp.int32) for _ in range(NBUF)],
            [pltpu.VMEM((CHUNK,), jnp.int32) for _ in range(NBUF)],
            [pltpu.VMEM((max(tailn, 1),), jnp.int32) for _ in range(2)],
            [pltpu.VMEM((CHUNK, d), jnp.float32) for _ in range(NBUF)],
            pltpu.VMEM_SHARED((n, d), jnp.float32),
            [pltpu.SemaphoreType.DMA for _ in range(NBUF)],
            [pltpu.SemaphoreType.DMA for _ in range(NBUF)],
            pltpu.SemaphoreType.DMA,
        ],
    )
    def sc_agg(h_hbm, eidx_hbm, zero_hbm, out_hbm,
               sidx, didx, tidx, rows, acc, isems, gsems, ssem):
        cid = lax.axis_index("c")
        sid = lax.axis_index("s")
        wid = cid * NS + sid
        base = wid * per_w

        def fetch_idx(c, b):
            pltpu.async_copy(
                eidx_hbm.at[0, pl.ds(base + c * CHUNK, CHUNK)], sidx[b],
                isems[b])
            pltpu.async_copy(
                eidx_hbm.at[1, pl.ds(base + c * CHUNK, CHUNK)], didx[b],
                isems[b])

        def wait_idx(c, b):
            pltpu.make_async_copy(
                eidx_hbm.at[0, pl.ds(base + c * CHUNK, CHUNK)], sidx[b],
                isems[b]).wait()
            pltpu.make_async_copy(
                eidx_hbm.at[1, pl.ds(base + c * CHUNK, CHUNK)], didx[b],
                isems[b]).wait()

        for b in range(NBUF):
            fetch_idx(b, b)
        for t in range(max_per_tile):
            c = sid + t * NS

            @pl.when(c < nzch)
            def _(c=c):
                pltpu.async_copy(zero_hbm, acc.at[pl.ds(c * ZCH, ZCH)],
                                 ssem)

        for t in range(max_per_tile):
            c = sid + t * NS

            @pl.when(c < nzch)
            def _(c=c):
                pltpu.make_async_copy(
                    zero_hbm, acc.at[pl.ds(c * ZCH, ZCH)], ssem).wait()

        plsc.subcore_barrier()

        def group_body(t, carry):
            c0 = t * NBUF
            descs = []
            for b in range(NBUF):
                wait_idx(c0 + b, b)
                descs.append(pltpu.async_copy(
                    h_hbm.at[sidx[b]], rows[b], gsems[b]))
            sdescs = []
            for b in range(NBUF):
                descs[b].wait()
                sdescs.append(pltpu.async_copy(
                    rows[b], acc.at[didx[b]], ssem, add=True))
            for b in range(NBUF):
                sdescs[b].wait()
                cn = c0 + NBUF + b

                @pl.when(cn < chunks)
                def _(cn=cn, b=b):
                    fetch_idx(cn, b)

            return carry

        lax.fori_loop(0, chunks // NBUF, group_body, 0)
        if tailn:
            toff = base + chunks * CHUNK
            pltpu.async_copy(eidx_hbm.at[0, pl.ds(toff, tailn)], tidx[0],
                             isems[0])
            pltpu.async_copy(eidx_hbm.at[1, pl.ds(toff, tailn)], tidx[1],
                             isems[0])
            pltpu.make_async_copy(eidx_hbm.at[0, pl.ds(toff, tailn)], tidx[0],
                                  isems[0]).wait()
            pltpu.make_async_copy(eidx_hbm.at[1, pl.ds(toff, tailn)], tidx[1],
                                  isems[0]).wait()
            trows = rows[0].at[pl.ds(0, tailn)]
            pltpu.async_copy(h_hbm.at[tidx[0]], trows, gsems[0]).wait()
            pltpu.sync_copy(trows, acc.at[tidx[1]], add=True)
        plsc.subcore_barrier()

        for t in range(max_per_tile):
            c = sid + t * NS

            @pl.when(c < nzch)
            def _(c=c):
                pltpu.async_copy(acc.at[pl.ds(c * ZCH, ZCH)],
                                 out_hbm.at[cid, pl.ds(c * ZCH, ZCH)],
                                 ssem)

        for t in range(max_per_tile):
            c = sid + t * NS

            @pl.when(c < nzch)
            def _(c=c):
                pltpu.make_async_copy(
                    acc.at[pl.ds(c * ZCH, ZCH)],
                    out_hbm.at[cid, pl.ds(c * ZCH, ZCH)], ssem).wait()

    return sc_agg


def _bn_body(n_rows, p0_ref, p1_ref, b_ref, gamma_ref, beta_ref, out_ref,
             agg_buf, stat_buf):
    k = pl.program_id(0)
    i = pl.program_id(1)

    @pl.when(k == 0)
    def _():
        agg = p0_ref[0] + p1_ref[0] + b_ref[...]
        agg_buf[i] = agg

        @pl.when(i == 0)
        def _():
            stat_buf[...] = jnp.zeros_like(stat_buf)

        stat_buf[0, :] += jnp.sum(agg, axis=0)
        stat_buf[1, :] += jnp.sum(agg * agg, axis=0)

    @pl.when(k == 1)
    def _():
        mean = stat_buf[0:1, :] * (1.0 / n_rows)
        ex2 = stat_buf[1:2, :] * (1.0 / n_rows)
        var = ex2 - mean * mean
        scale = jax.lax.rsqrt(var + EPS) * gamma_ref[...]
        out_ref[...] = (agg_buf[i] - mean) * scale + beta_ref[...]


def kernel(x, edge_index, W, b, gamma, beta):
    n, d = x.shape
    e = edge_index.shape[1]
    block_rows = 1000

    h = _matmul(x, W, block_rows)

    zeros = jnp.zeros((ZCH, d), jnp.float32)
    partial = _make_sc_agg(n, d, e)(h, edge_index, zeros)

    nblocks = n // block_rows
    b2 = b.reshape(1, d)
    out = pl.pallas_call(
        functools.partial(_bn_body, float(n)),
        grid=(2, nblocks),
        in_specs=[
            pl.BlockSpec((1, block_rows, d),
                         lambda k, i: (0, jnp.where(k == 0, i, 0), 0)),
            pl.BlockSpec((1, block_rows, d),
                         lambda k, i: (1, jnp.where(k == 0, i, 0), 0)),
            pl.BlockSpec((1, d), lambda k, i: (0, 0)),
            pl.BlockSpec((1, d), lambda k, i: (0, 0)),
            pl.BlockSpec((1, d), lambda k, i: (0, 0)),
        ],
        out_specs=pl.BlockSpec((block_rows, d),
                               lambda k, i: (jnp.where(k == 0, 0, i), 0)),
        out_shape=jax.ShapeDtypeStruct((n, d), jnp.float32),
        scratch_shapes=[
            pltpu.VMEM((nblocks, block_rows, d), jnp.float32),
            pltpu.VMEM((8, d), jnp.float32),
        ],
    )(partial, partial, b2, gamma.reshape(1, d), beta.reshape(1, d))

    return out


# speedup vs baseline: 1.1795x; 1.1795x over previous
import functools

import jax
import jax.numpy as jnp
from jax import lax
from jax.experimental import pallas as pl
from jax.experimental.pallas import tpu as pltpu
from jax.experimental.pallas import tpu_sc as plsc

EPS = 1e-5

NC = 2
NS = 16
CHUNK = 128
NBUF = 3
ZCH = 80


def _matmul_body(x_ref, w_ref, h_ref):
    h_ref[...] = jnp.dot(x_ref[...], w_ref[...],
                         preferred_element_type=jnp.float32
                         ).astype(jnp.bfloat16)


def _matmul(x, W, block_rows):
    n, d = x.shape
    return pl.pallas_call(
        _matmul_body,
        grid=(n // block_rows,),
        in_specs=[
            pl.BlockSpec((block_rows, d), lambda i: (i, 0)),
            pl.BlockSpec((d, d), lambda i: (0, 0)),
        ],
        out_specs=pl.BlockSpec((block_rows, d), lambda i: (i, 0)),
        out_shape=jax.ShapeDtypeStruct((n, d), jnp.bfloat16),
    )(x, W)


def _make_sc_agg(n, d, e):
    per_w = e // (NC * NS)
    chunks = per_w // CHUNK
    tailn = per_w - chunks * CHUNK
    assert tailn % 8 == 0 and chunks % NBUF == 0
    nzch = n // ZCH
    max_per_tile = (nzch + NS - 1) // NS

    mesh = plsc.VectorSubcoreMesh(core_axis_name="c", subcore_axis_name="s")

    @functools.partial(
        pl.kernel,
        mesh=mesh,
        out_type=jax.ShapeDtypeStruct((NC, n, d), jnp.bfloat16),
        compiler_params=pltpu.CompilerParams(use_tc_tiling_on_sc=False),
        scratch_types=[
            [pltpu.VMEM((CHUNK,), jnp.int32) for _ in range(NBUF)],
            [pltpu.VMEM((CHUNK,), jnp.int32) for _ in range(NBUF)],
            [pltpu.VMEM((max(tailn, 1),), jnp.int32) for _ in range(2)],
            [pltpu.VMEM((CHUNK, d), jnp.bfloat16) for _ in range(NBUF)],
            pltpu.VMEM_SHARED((n, d), jnp.bfloat16),
            [pltpu.SemaphoreType.DMA for _ in range(NBUF)],
            [pltpu.SemaphoreType.DMA for _ in range(NBUF)],
            pltpu.SemaphoreType.DMA,
        ],
    )
    def sc_agg(h_hbm, eidx_hbm, zero_hbm, out_hbm,
               sidx, didx, tidx, rows, acc, isems, gsems, ssem):
        cid = lax.axis_index("c")
        sid = lax.axis_index("s")
        wid = cid * NS + sid
        base = wid * per_w

        def fetch_idx(c, b):
            pltpu.async_copy(
                eidx_hbm.at[0, pl.ds(base + c * CHUNK, CHUNK)], sidx[b],
                isems[b])
            pltpu.async_copy(
                eidx_hbm.at[1, pl.ds(base + c * CHUNK, CHUNK)], didx[b],
                isems[b])

        def wait_idx(c, b):
            pltpu.make_async_copy(
                eidx_hbm.at[0, pl.ds(base + c * CHUNK, CHUNK)], sidx[b],
                isems[b]).wait()
            pltpu.make_async_copy(
                eidx_hbm.at[1, pl.ds(base + c * CHUNK, CHUNK)], didx[b],
                isems[b]).wait()

        for b in range(NBUF):
            fetch_idx(b, b)
        for t in range(max_per_tile):
            c = sid + t * NS

            @pl.when(c < nzch)
            def _(c=c):
                pltpu.async_copy(zero_hbm, acc.at[pl.ds(c * ZCH, ZCH)],
                                 ssem)

        for t in range(max_per_tile):
            c = sid + t * NS

            @pl.when(c < nzch)
            def _(c=c):
                pltpu.make_async_copy(
                    zero_hbm, acc.at[pl.ds(c * ZCH, ZCH)], ssem).wait()

        plsc.subcore_barrier()

        def group_body(t, carry):
            c0 = t * NBUF
            descs = []
            for b in range(NBUF):
                wait_idx(c0 + b, b)
                descs.append(pltpu.async_copy(
                    h_hbm.at[sidx[b]], rows[b], gsems[b]))
            sdescs = []
            for b in range(NBUF):
                descs[b].wait()
                sdescs.append(pltpu.async_copy(
                    rows[b], acc.at[didx[b]], ssem, add=True))
            for b in range(NBUF):
                sdescs[b].wait()
                cn = c0 + NBUF + b

                @pl.when(cn < chunks)
                def _(cn=cn, b=b):
                    fetch_idx(cn, b)

            return carry

        lax.fori_loop(0, chunks // NBUF, group_body, 0)
        if tailn:
            toff = base + chunks * CHUNK
            pltpu.async_copy(eidx_hbm.at[0, pl.ds(toff, tailn)], tidx[0],
                             isems[0])
            pltpu.async_copy(eidx_hbm.at[1, pl.ds(toff, tailn)], tidx[1],
                             isems[0])
            pltpu.make_async_copy(eidx_hbm.at[0, pl.ds(toff, tailn)], tidx[0],
                                  isems[0]).wait()
            pltpu.make_async_copy(eidx_hbm.at[1, pl.ds(toff, tailn)], tidx[1],
                                  isems[0]).wait()
            trows = rows[0].at[pl.ds(0, tailn)]
            pltpu.async_copy(h_hbm.at[tidx[0]], trows, gsems[0]).wait()
            pltpu.sync_copy(trows, acc.at[tidx[1]], add=True)
        plsc.subcore_barrier()

        for t in range(max_per_tile):
            c = sid + t * NS

            @pl.when(c < nzch)
            def _(c=c):
                pltpu.async_copy(acc.at[pl.ds(c * ZCH, ZCH)],
                                 out_hbm.at[cid, pl.ds(c * ZCH, ZCH)],
                                 ssem)

        for t in range(max_per_tile):
            c = sid + t * NS

            @pl.when(c < nzch)
            def _(c=c):
                pltpu.make_async_copy(
                    acc.at[pl.ds(c * ZCH, ZCH)],
                    out_hbm.at[cid, pl.ds(c * ZCH, ZCH)], ssem).wait()

    return sc_agg


def _bn_body(n_rows, p0_ref, p1_ref, b_ref, gamma_ref, beta_ref, out_ref,
             agg_buf, stat_buf):
    k = pl.program_id(0)
    i = pl.program_id(1)

    @pl.when(k == 0)
    def _():
        agg = (p0_ref[0].astype(jnp.float32) +
               p1_ref[0].astype(jnp.float32) + b_ref[...])
        agg_buf[i] = agg

        @pl.when(i == 0)
        def _():
            stat_buf[...] = jnp.zeros_like(stat_buf)

        stat_buf[0, :] += jnp.sum(agg, axis=0)
        stat_buf[1, :] += jnp.sum(agg * agg, axis=0)

    @pl.when(k == 1)
    def _():
        mean = stat_buf[0:1, :] * (1.0 / n_rows)
        ex2 = stat_buf[1:2, :] * (1.0 / n_rows)
        var = ex2 - mean * mean
        scale = jax.lax.rsqrt(var + EPS) * gamma_ref[...]
        out_ref[...] = (agg_buf[i] - mean) * scale + beta_ref[...]


def kernel(x, edge_index, W, b, gamma, beta):
    n, d = x.shape
    e = edge_index.shape[1]
    block_rows = 1000

    h = _matmul(x, W, block_rows)

    zeros = jnp.zeros((ZCH, d), jnp.bfloat16)
    partial = _make_sc_agg(n, d, e)(h, edge_index, zeros)

    nblocks = n // block_rows
    b2 = b.reshape(1, d)
    out = pl.pallas_call(
        functools.partial(_bn_body, float(n)),
        grid=(2, nblocks),
        in_specs=[
            pl.BlockSpec((1, block_rows, d),
                         lambda k, i: (0, jnp.where(k == 0, i, 0), 0)),
            pl.BlockSpec((1, block_rows, d),
                         lambda k, i: (1, jnp.where(k == 0, i, 0), 0)),
            pl.BlockSpec((1, d), lambda k, i: (0, 0)),
            pl.BlockSpec((1, d), lambda k, i: (0, 0)),
            pl.BlockSpec((1, d), lambda k, i: (0, 0)),
        ],
        out_specs=pl.BlockSpec((block_rows, d),
                               lambda k, i: (jnp.where(k == 0, 0, i), 0)),
        out_shape=jax.ShapeDtypeStruct((n, d), jnp.float32),
        scratch_shapes=[
            pltpu.VMEM((nblocks, block_rows, d), jnp.float32),
            pltpu.VMEM((8, d), jnp.float32),
        ],
    )(partial, partial, b2, gamma.reshape(1, d), beta.reshape(1, d))

    return out
